# baseline (speedup 1.0000x reference)
import functools

import jax
import jax.numpy as jnp
from jax import lax
from jax.experimental import pallas as pl
from jax.experimental.pallas import tpu as pltpu
from jax.experimental.pallas import tpu_sc as plsc

_NC = 2
_NS = 16
_NW = _NC * _NS
_IDX_CHUNK = 128


def _sc_gather_body(uidx, iidx, gu_t, gi_t, mu_t, mi_t,
                    out_gu, out_gi, out_mu, out_mi,
                    uv, iv, guv, giv, muv, miv, sem):
    bpw = uv.shape[0] * _IDX_CHUNK
    wid = lax.axis_index("c") * _NS + lax.axis_index("s")
    base = wid * bpw
    pltpu.sync_copy(uidx.at[wid], uv)
    pltpu.sync_copy(iidx.at[wid], iv)
    copies = []
    for tbl, idxv, dst in ((gu_t, uv, guv), (gi_t, iv, giv),
                           (mu_t, uv, muv), (mi_t, iv, miv)):
        for j in range(uv.shape[0]):
            copies.append(pltpu.async_copy(
                tbl.at[idxv.at[j]], dst.at[pl.ds(j * _IDX_CHUNK, _IDX_CHUNK)],
                sem))
    for c in copies:
        c.wait()
    pltpu.sync_copy(guv, out_gu.at[pl.ds(base, bpw)])
    pltpu.sync_copy(giv, out_gi.at[pl.ds(base, bpw)])
    pltpu.sync_copy(muv, out_mu.at[pl.ds(base, bpw)])
    pltpu.sync_copy(miv, out_mi.at[pl.ds(base, bpw)])


def _sc_gather(uidx, iidx, gu_t, gi_t, mu_t, mi_t):
    batch = uidx.shape[0] * uidx.shape[1] * uidx.shape[2]
    bpw = batch // _NW
    ch = uidx.shape[1]
    dim = gu_t.shape[1]
    row = jax.ShapeDtypeStruct((batch, dim), jnp.float32)
    gather = pl.kernel(
        _sc_gather_body,
        out_type=(row, row, row, row),
        mesh=plsc.VectorSubcoreMesh(core_axis_name="c", subcore_axis_name="s"),
        scratch_types=[
            pltpu.VMEM((ch, _IDX_CHUNK), jnp.int32),
            pltpu.VMEM((ch, _IDX_CHUNK), jnp.int32),
            pltpu.VMEM((bpw, dim), jnp.float32),
            pltpu.VMEM((bpw, dim), jnp.float32),
            pltpu.VMEM((bpw, dim), jnp.float32),
            pltpu.VMEM((bpw, dim), jnp.float32),
            pltpu.SemaphoreType.DMA,
        ],
        compiler_params=pltpu.CompilerParams(use_tc_tiling_on_sc=False),
    )
    return gather(uidx, iidx, gu_t, gi_t, mu_t, mi_t)


def _mlp_body(gu_ref, gi_ref, mu_ref, mi_ref, w1a_ref, w1b_ref, b1_ref,
              w2_ref, b2_ref, w3_ref, b3_ref, wog_ref, woh_ref, bo_ref,
              out_ref):
    f32 = jnp.float32
    h = jnp.dot(mu_ref[...], w1a_ref[...], preferred_element_type=f32)
    h += jnp.dot(mi_ref[...], w1b_ref[...], preferred_element_type=f32)
    h = jnp.maximum(h + b1_ref[...], 0.0)
    h = jnp.maximum(
        jnp.dot(h, w2_ref[...], preferred_element_type=f32) + b2_ref[...], 0.0)
    h = jnp.maximum(
        jnp.dot(h, w3_ref[...], preferred_element_type=f32) + b3_ref[...], 0.0)
    gmf = gu_ref[...] * gi_ref[...]
    logit = (jnp.sum(gmf * wog_ref[...], axis=1)
             + jnp.sum(h * woh_ref[...], axis=1) + bo_ref[0, 0])
    out_ref[...] = jax.nn.sigmoid(logit)


def _mlp(gu, gi, mu, mi, W1, b1, W2, b2, W3, b3, Wo, bo, block_rows):
    batch, mdim = mu.shape
    gdim = gu.shape[1]
    w1a = W1[:mdim]
    w1b = W1[mdim:]
    wog = Wo[:gdim].reshape(1, gdim)
    woh = Wo[gdim:].reshape(1, Wo.shape[0] - gdim)
    row_spec = pl.BlockSpec((block_rows, mdim), lambda i: (i, 0))
    full = lambda a: pl.BlockSpec(a.shape, lambda i: (0,) * a.ndim)
    args = (gu, gi, mu, mi, w1a, w1b, b1.reshape(1, -1), W2,
            b2.reshape(1, -1), W3, b3.reshape(1, -1), wog, woh,
            bo.reshape(1, 1))
    in_specs = [row_spec, row_spec, row_spec, row_spec] + [
        full(a) for a in args[4:]]
    return pl.pallas_call(
        _mlp_body,
        grid=(batch // block_rows,),
        in_specs=in_specs,
        out_specs=pl.BlockSpec((block_rows,), lambda i: (i,)),
        out_shape=jax.ShapeDtypeStruct((batch,), jnp.float32),
    )(*args)


def kernel(user_indices, item_indices, gmf_user_table, gmf_item_table,
           mlp_user_table, mlp_item_table, W1, b1, W2, b2, W3, b3, Wo, bo):
    batch = user_indices.shape[0]
    ch = batch // (_NW * _IDX_CHUNK)
    uidx = user_indices.reshape(_NW, ch, _IDX_CHUNK)
    iidx = item_indices.reshape(_NW, ch, _IDX_CHUNK)
    gu, gi, mu, mi = _sc_gather(uidx, iidx, gmf_user_table, gmf_item_table,
                                mlp_user_table, mlp_item_table)
    return _mlp(gu, gi, mu, mi, W1, b1, W2, b2, W3, b3, Wo, bo,
                block_rows=2048)
